# baseline (speedup 1.0000x reference)
import functools

import jax
import jax.numpy as jnp
import numpy as np
from jax import lax
from jax.experimental import pallas as pl
from jax.experimental.pallas import tpu as pltpu, tpu_sc as plsc


def _make_ebag(V, T, D, NB, L, NC, NS):
    NW = NC * NS
    BAGS_W = NB // NW
    G_BAGS = 104
    GROUPS = BAGS_W // G_BAGS
    GE = G_BAGS * L
    GR = GE // D
    CH = 80
    ROWS_G = GE // CH
    BAGS_CH = CH // L
    CHUNKS_W = BAGS_W * L // CH
    NBUF = 13
    OUTER = CHUNKS_W // NBUF
    N = NB * L
    IDX_ROW0 = V * T
    TBL_ROW0 = IDX_ROW0 + N // D

    mesh = plsc.VectorSubcoreMesh(core_axis_name="c", subcore_axis_name="s")

    @functools.partial(
        pl.kernel,
        out_type=jax.ShapeDtypeStruct((NB, D), jnp.float32),
        mesh=mesh,
        scratch_types=[
            pltpu.VMEM((GR, D), jnp.float32),
            pltpu.VMEM((GE,), jnp.int32),
            pltpu.VMEM((2, GR, D), jnp.float32),
            pltpu.VMEM((CHUNKS_W, CH), jnp.int32),
            pltpu.VMEM((NBUF, CH, D), jnp.float32),
            pltpu.VMEM((NBUF, BAGS_CH, D), jnp.float32),
            [pltpu.SemaphoreType.DMA] * NBUF,
            [pltpu.SemaphoreType.DMA] * NBUF,
            [pltpu.SemaphoreType.DMA] * 2,
        ],
        compiler_params=pltpu.CompilerParams(use_tc_tiling_on_sc=False),
    )
    def ebag(big_hbm, out_hbm,
             tblf_v, tbli_v, raw_v, flat_v, rows_v, out_v, gsem, osem, rsem):
        wid = lax.axis_index("s") * NC + lax.axis_index("c")
        w_row = wid * (BAGS_W * L // D)
        w_bag = wid * BAGS_W

        pltpu.sync_copy(big_hbm.at[pl.ds(TBL_ROW0, GR)], tblf_v)

        def tbl_body(rr, c2):
            for s2 in range(D // 16):
                tbli_v[pl.ds(rr * D + s2 * 16, 16)] = (
                    tblf_v[rr, pl.ds(s2 * 16, 16)].astype(jnp.int32))
            return c2

        lax.fori_loop(0, GR, tbl_body, 0)

        def raw_copy(g, par):
            return pltpu.make_async_copy(
                big_hbm.at[pl.ds(IDX_ROW0 + w_row + g * GR, GR)],
                raw_v.at[par], rsem[par])

        raw_copy(0, 0).start()

        def group_body(gg, carry):
            for par in range(2):
                g = gg * 2 + par

                @pl.when(g + 1 < GROUPS)
                def _():
                    raw_copy(g + 1, 1 - par).start()

                raw_copy(g, par).wait()

                def idx_body(r, c2):
                    e = r * CH
                    for s in range(CH // 16):
                        es = e + s * 16
                        raw = raw_v[par, es // D,
                                    pl.ds(es % D, 16)].astype(jnp.int32)
                        tbl = tbli_v[pl.ds(es, 16)]
                        flat_v[g * ROWS_G + r, pl.ds(s * 16, 16)] = raw * T + tbl
                    return c2

                lax.fori_loop(0, ROWS_G, idx_body, 0)
            return carry

        lax.fori_loop(0, GROUPS // 2, group_body, 0)

        def gather(c, b):
            return pltpu.make_async_copy(
                big_hbm.at[flat_v.at[c]], rows_v.at[b], gsem[b])

        def out_copy(c, b):
            return pltpu.make_async_copy(
                out_v.at[b], out_hbm.at[pl.ds(w_bag + c * BAGS_CH, BAGS_CH)],
                osem[b])

        for b in range(NBUF):
            gather(b, b).start()

        def outer_body(c0, carry):
            cb = c0 * NBUF
            for b in range(NBUF):
                c = cb + b
                gather(c, b).wait()

                @pl.when(c0 > 0)
                def _():
                    out_copy(c, b).wait()

                for k in range(BAGS_CH):
                    base = k * L
                    for h in range(D // 16):
                        s = pl.ds(h * 16, 16)
                        a0 = rows_v[b, base, s] + rows_v[b, base + 1, s]
                        a1 = rows_v[b, base + 2, s] + rows_v[b, base + 3, s]
                        a2 = rows_v[b, base + 4, s] + rows_v[b, base + 5, s]
                        a3 = rows_v[b, base + 6, s] + rows_v[b, base + 7, s]
                        a4 = rows_v[b, base + 8, s] + rows_v[b, base + 9, s]
                        a5 = rows_v[b, base + 10, s] + rows_v[b, base + 11, s]
                        a6 = rows_v[b, base + 12, s] + rows_v[b, base + 13, s]
                        a7 = rows_v[b, base + 14, s] + rows_v[b, base + 15, s]
                        a8 = rows_v[b, base + 16, s] + rows_v[b, base + 17, s]
                        a9 = rows_v[b, base + 18, s] + rows_v[b, base + 19, s]
                        b0 = a0 + a1
                        b1 = a2 + a3
                        b2 = a4 + a5
                        b3 = a6 + a7
                        b4 = a8 + a9
                        out_v[b, k, s] = ((b0 + b1) + (b2 + b3)) + b4

                out_copy(c, b).start()

                @pl.when(c + NBUF < CHUNKS_W)
                def _():
                    gather(c + NBUF, b).start()
            return carry

        lax.fori_loop(0, OUTER, outer_body, 0)

        for b in range(NBUF):
            out_copy(CHUNKS_W - NBUF + b, b).wait()

    return ebag


def kernel(embedding_weights, sharded_sparse_features, sharded_offsets):
    V, T, D = embedding_weights.shape
    N = sharded_sparse_features.shape[0]
    NB = sharded_offsets.shape[0] - 1
    L = N // NB
    info = plsc.get_sparse_core_info()
    ebag = _make_ebag(V, T, D, NB, L, info.num_cores, info.num_subcores)
    ge = 104 * L
    tbl_pat = np.tile(np.repeat(np.arange(T, dtype=np.float32), L),
                      ge // (T * L)).reshape(ge // D, D)
    big = jnp.concatenate([
        embedding_weights.reshape(V * T, D),
        sharded_sparse_features.astype(jnp.float32).reshape(N // D, D),
        jnp.asarray(tbl_pat),
    ], axis=0)
    out = ebag(big)
    return out.reshape(NB // T, T, D)


# speedup vs baseline: 2.4513x; 2.4513x over previous
import functools

import jax
import jax.numpy as jnp
import numpy as np
from jax import lax
from jax.experimental import pallas as pl
from jax.experimental.pallas import tpu as pltpu, tpu_sc as plsc


def _make_ebag(V, T, D, NB, L, NC, NS):
    NW = NC * NS
    PERIOD = T * L
    IR = 128
    BAGS_W = NB // NW
    G_BAGS = 416
    GROUPS = BAGS_W // G_BAGS
    GE = G_BAGS * L
    GR = GE // IR
    CH = 80
    ROWS_G = GE // CH
    BAGS_CH = CH // L
    CHUNKS_W = BAGS_W * L // CH
    NBUF = 13
    OUTER = CHUNKS_W // NBUF

    mesh = plsc.VectorSubcoreMesh(core_axis_name="c", subcore_axis_name="s")

    @functools.partial(
        pl.kernel,
        out_type=jax.ShapeDtypeStruct((NB, D), jnp.float32),
        mesh=mesh,
        scratch_types=[
            pltpu.VMEM((PERIOD + 16,), jnp.int32),
            pltpu.VMEM((2, GR, IR), jnp.int32),
            pltpu.VMEM((CHUNKS_W, CH), jnp.int32),
            pltpu.VMEM((NBUF, CH, D), jnp.float32),
            pltpu.VMEM((NBUF, BAGS_CH, D), jnp.float32),
            [pltpu.SemaphoreType.DMA] * NBUF,
            [pltpu.SemaphoreType.DMA] * NBUF,
            [pltpu.SemaphoreType.DMA] * 2,
        ],
        compiler_params=pltpu.CompilerParams(use_tc_tiling_on_sc=False),
    )
    def ebag(table_hbm, idx_hbm, tbl_hbm, out_hbm,
             tbl_v, raw_v, flat_v, rows_v, out_v, gsem, osem, rsem):
        wid = lax.axis_index("s") * NC + lax.axis_index("c")
        w_row = wid * (BAGS_W * L // IR)
        w_bag = wid * BAGS_W

        pltpu.sync_copy(tbl_hbm, tbl_v)

        def raw_copy(g, par):
            return pltpu.make_async_copy(
                idx_hbm.at[pl.ds(w_row + g * GR, GR)], raw_v.at[par], rsem[par])

        raw_copy(0, 0).start()

        def group_body(gg, carry):
            for par in range(2):
                g = gg * 2 + par

                @pl.when(g + 1 < GROUPS)
                def _():
                    raw_copy(g + 1, 1 - par).start()

                raw_copy(g, par).wait()

                def idx_body(r, c2):
                    e = r * CH
                    for s in range(CH // 16):
                        es = e + s * 16
                        raw = raw_v[par, es // IR, pl.ds(es % IR, 16)]
                        tbl = tbl_v[pl.ds(es % PERIOD, 16)]
                        flat_v[g * ROWS_G + r, pl.ds(s * 16, 16)] = raw * T + tbl
                    return c2

                lax.fori_loop(0, ROWS_G, idx_body, 0)
            return carry

        lax.fori_loop(0, GROUPS // 2, group_body, 0)

        def gather(c, b):
            return pltpu.make_async_copy(
                table_hbm.at[flat_v.at[c]], rows_v.at[b], gsem[b])

        def out_copy(c, b):
            return pltpu.make_async_copy(
                out_v.at[b], out_hbm.at[pl.ds(w_bag + c * BAGS_CH, BAGS_CH)],
                osem[b])

        for b in range(NBUF):
            gather(b, b).start()

        def outer_body(c0, carry):
            cb = c0 * NBUF
            for b in range(NBUF):
                c = cb + b
                gather(c, b).wait()

                @pl.when(c0 > 0)
                def _():
                    out_copy(c, b).wait()

                for k in range(BAGS_CH):
                    base = k * L
                    for h in range(D // 16):
                        s = pl.ds(h * 16, 16)
                        a0 = rows_v[b, base, s] + rows_v[b, base + 1, s]
                        a1 = rows_v[b, base + 2, s] + rows_v[b, base + 3, s]
                        a2 = rows_v[b, base + 4, s] + rows_v[b, base + 5, s]
                        a3 = rows_v[b, base + 6, s] + rows_v[b, base + 7, s]
                        a4 = rows_v[b, base + 8, s] + rows_v[b, base + 9, s]
                        a5 = rows_v[b, base + 10, s] + rows_v[b, base + 11, s]
                        a6 = rows_v[b, base + 12, s] + rows_v[b, base + 13, s]
                        a7 = rows_v[b, base + 14, s] + rows_v[b, base + 15, s]
                        a8 = rows_v[b, base + 16, s] + rows_v[b, base + 17, s]
                        a9 = rows_v[b, base + 18, s] + rows_v[b, base + 19, s]
                        b0 = a0 + a1
                        b1 = a2 + a3
                        b2 = a4 + a5
                        b3 = a6 + a7
                        b4 = a8 + a9
                        out_v[b, k, s] = ((b0 + b1) + (b2 + b3)) + b4

                out_copy(c, b).start()

                @pl.when(c + NBUF < CHUNKS_W)
                def _():
                    gather(c + NBUF, b).start()
            return carry

        lax.fori_loop(0, OUTER, outer_body, 0)

        for b in range(NBUF):
            out_copy(CHUNKS_W - NBUF + b, b).wait()

    return ebag


def kernel(embedding_weights, sharded_sparse_features, sharded_offsets):
    V, T, D = embedding_weights.shape
    N = sharded_sparse_features.shape[0]
    NB = sharded_offsets.shape[0] - 1
    L = N // NB
    info = plsc.get_sparse_core_info()
    ebag = _make_ebag(V, T, D, NB, L, info.num_cores, info.num_subcores)
    table = embedding_weights.reshape(V * T, D)
    idx2 = sharded_sparse_features.reshape(N // 128, 128)
    pat = np.repeat(np.arange(T, dtype=np.int32), L)
    tbl_pat = jnp.asarray(np.concatenate([pat, pat[:16]]))
    out = ebag(table, idx2, tbl_pat)
    return out.reshape(NB // T, T, D)


# speedup vs baseline: 2.4817x; 1.0124x over previous
import functools

import jax
import jax.numpy as jnp
from jax import lax
from jax.experimental import pallas as pl
from jax.experimental.pallas import tpu as pltpu, tpu_sc as plsc


def _make_ebag(V, T, D, NB, L, NC, NS):
    NW = NC * NS
    BAGS_W = NB // NW
    CH = 80
    BAGS_CH = CH // L
    CHUNKS_W = BAGS_W * L // CH
    NBUF = 13
    OUTER = CHUNKS_W // NBUF

    mesh = plsc.VectorSubcoreMesh(core_axis_name="c", subcore_axis_name="s")

    @functools.partial(
        pl.kernel,
        out_type=jax.ShapeDtypeStruct((NB, D), jnp.float32),
        mesh=mesh,
        scratch_types=[
            pltpu.VMEM((CHUNKS_W, CH), jnp.int32),
            pltpu.VMEM((NBUF, CH, D), jnp.float32),
            pltpu.VMEM((NBUF, BAGS_CH, D), jnp.float32),
            [pltpu.SemaphoreType.DMA] * NBUF,
            [pltpu.SemaphoreType.DMA] * NBUF,
        ],
        compiler_params=pltpu.CompilerParams(use_tc_tiling_on_sc=False),
    )
    def ebag(table_hbm, flat_hbm, out_hbm,
             flat_v, rows_v, out_v, gsem, osem):
        wid = lax.axis_index("s") * NC + lax.axis_index("c")
        w_bag = wid * BAGS_W

        pltpu.sync_copy(flat_hbm.at[pl.ds(wid * CHUNKS_W, CHUNKS_W)], flat_v)

        def gather(c, b):
            return pltpu.make_async_copy(
                table_hbm.at[flat_v.at[c]], rows_v.at[b], gsem[b])

        def out_copy(c, b):
            return pltpu.make_async_copy(
                out_v.at[b], out_hbm.at[pl.ds(w_bag + c * BAGS_CH, BAGS_CH)],
                osem[b])

        for b in range(NBUF):
            gather(b, b).start()

        def outer_body(c0, carry):
            cb = c0 * NBUF
            for b in range(NBUF):
                c = cb + b
                gather(c, b).wait()

                @pl.when(c0 > 0)
                def _():
                    out_copy(c, b).wait()

                for k in range(BAGS_CH):
                    base = k * L
                    for h in range(D // 16):
                        s = pl.ds(h * 16, 16)
                        a0 = rows_v[b, base, s] + rows_v[b, base + 1, s]
                        a1 = rows_v[b, base + 2, s] + rows_v[b, base + 3, s]
                        a2 = rows_v[b, base + 4, s] + rows_v[b, base + 5, s]
                        a3 = rows_v[b, base + 6, s] + rows_v[b, base + 7, s]
                        a4 = rows_v[b, base + 8, s] + rows_v[b, base + 9, s]
                        a5 = rows_v[b, base + 10, s] + rows_v[b, base + 11, s]
                        a6 = rows_v[b, base + 12, s] + rows_v[b, base + 13, s]
                        a7 = rows_v[b, base + 14, s] + rows_v[b, base + 15, s]
                        a8 = rows_v[b, base + 16, s] + rows_v[b, base + 17, s]
                        a9 = rows_v[b, base + 18, s] + rows_v[b, base + 19, s]
                        b0 = a0 + a1
                        b1 = a2 + a3
                        b2 = a4 + a5
                        b3 = a6 + a7
                        b4 = a8 + a9
                        out_v[b, k, s] = ((b0 + b1) + (b2 + b3)) + b4

                out_copy(c, b).start()

                @pl.when(c + NBUF < CHUNKS_W)
                def _():
                    gather(c + NBUF, b).start()
            return carry

        lax.fori_loop(0, OUTER, outer_body, 0)

        for b in range(NBUF):
            out_copy(CHUNKS_W - NBUF + b, b).wait()

    return ebag


def kernel(embedding_weights, sharded_sparse_features, sharded_offsets):
    V, T, D = embedding_weights.shape
    N = sharded_sparse_features.shape[0]
    NB = sharded_offsets.shape[0] - 1
    L = N // NB
    info = plsc.get_sparse_core_info()
    ebag = _make_ebag(V, T, D, NB, L, info.num_cores, info.num_subcores)
    table = embedding_weights.reshape(V * T, D)
    tblid = (jnp.arange(N, dtype=jnp.int32) // L) % T
    flat2 = (sharded_sparse_features * T + tblid).reshape(N // 80, 80)
    out = ebag(table, flat2)
    return out.reshape(NB // T, T, D)


# speedup vs baseline: 2.4880x; 1.0025x over previous
import functools

import jax
import jax.numpy as jnp
from jax import lax
from jax.experimental import pallas as pl
from jax.experimental.pallas import tpu as pltpu, tpu_sc as plsc


def _make_ebag(V, T, D, NB, L, NC, NS):
    NW = NC * NS
    BAGS_W = NB // NW
    CH = 80
    BAGS_CH = CH // L
    CHUNKS_W = BAGS_W * L // CH
    NBUF = 13
    OUTER = CHUNKS_W // NBUF

    mesh = plsc.VectorSubcoreMesh(core_axis_name="c", subcore_axis_name="s")

    @functools.partial(
        pl.kernel,
        out_type=jax.ShapeDtypeStruct((NB, D), jnp.float32),
        mesh=mesh,
        scratch_types=[
            pltpu.VMEM((CHUNKS_W, CH), jnp.int32),
            pltpu.VMEM((NBUF, CH, D), jnp.float32),
            pltpu.VMEM((NBUF, BAGS_CH, D), jnp.float32),
            [pltpu.SemaphoreType.DMA] * NBUF,
            [pltpu.SemaphoreType.DMA] * NBUF,
        ],
        compiler_params=pltpu.CompilerParams(use_tc_tiling_on_sc=False),
    )
    def ebag(table_hbm, flat_hbm, out_hbm,
             flat_v, rows_v, out_v, gsem, osem):
        wid = lax.axis_index("s") * NC + lax.axis_index("c")
        w_bag = wid * BAGS_W

        pltpu.sync_copy(flat_hbm.at[pl.ds(wid * CHUNKS_W, CHUNKS_W)], flat_v)

        def gather(c, b):
            return pltpu.make_async_copy(
                table_hbm.at[flat_v.at[c]], rows_v.at[b], gsem[b])

        def out_copy(c, b):
            return pltpu.make_async_copy(
                out_v.at[b], out_hbm.at[pl.ds(w_bag + c * BAGS_CH, BAGS_CH)],
                osem[b])

        for b in range(NBUF):
            gather(b, b).start()

        def outer_body(c0, carry):
            cb = c0 * NBUF
            for b in range(NBUF):
                c = cb + b
                gather(c, b).wait()

                @pl.when(c0 > 0)
                def _():
                    out_copy(c, b).wait()

                for k in range(BAGS_CH):
                    base = k * L
                    for h in range(D // 16):
                        s = pl.ds(h * 16, 16)
                        a0 = rows_v[b, base, s] + rows_v[b, base + 1, s]
                        a1 = rows_v[b, base + 2, s] + rows_v[b, base + 3, s]
                        a2 = rows_v[b, base + 4, s] + rows_v[b, base + 5, s]
                        a3 = rows_v[b, base + 6, s] + rows_v[b, base + 7, s]
                        a4 = rows_v[b, base + 8, s] + rows_v[b, base + 9, s]
                        a5 = rows_v[b, base + 10, s] + rows_v[b, base + 11, s]
                        a6 = rows_v[b, base + 12, s] + rows_v[b, base + 13, s]
                        a7 = rows_v[b, base + 14, s] + rows_v[b, base + 15, s]
                        a8 = rows_v[b, base + 16, s] + rows_v[b, base + 17, s]
                        a9 = rows_v[b, base + 18, s] + rows_v[b, base + 19, s]
                        b0 = a0 + a1
                        b1 = a2 + a3
                        b2 = a4 + a5
                        b3 = a6 + a7
                        b4 = a8 + a9
                        out_v[b, k, s] = ((b0 + b1) + (b2 + b3)) + b4

                out_copy(c, b).start()

                @pl.when(c + NBUF < CHUNKS_W)
                def _():
                    gather(c + NBUF, b).start()
            return carry

        lax.fori_loop(0, OUTER, outer_body, 0)

        for b in range(NBUF):
            out_copy(CHUNKS_W - NBUF + b, b).wait()

    return ebag


def kernel(embedding_weights, sharded_sparse_features, sharded_offsets):
    V, T, D = embedding_weights.shape
    N = sharded_sparse_features.shape[0]
    NB = sharded_offsets.shape[0] - 1
    L = N // NB
    info = plsc.get_sparse_core_info()
    ebag = _make_ebag(V, T, D, NB, L, info.num_cores, info.num_subcores)
    table = embedding_weights.reshape(V * T, D)
    tblid = (jnp.arange(N, dtype=jnp.int32) // L) % T
    flat1 = lax.optimization_barrier(sharded_sparse_features * T + tblid)
    flat2 = flat1.reshape(N // 80, 80)
    out = ebag(table, flat2)
    return out.reshape(NB // T, T, D)
